# Initial kernel scaffold; baseline (speedup 1.0000x reference)
import jax
import jax.numpy as jnp
from jax.experimental import pallas as pl


def kernel(af, bf, fnf, fef, atom_edge_index, motif_edge_index, a2f_src, a2f_dst, atom_graph_ids, motif_graph_ids, labels, W_ae1, b_ae1, W_ae2, b_ae2, W_me1, b_me1, W_me2, b_me2, W_aen, b_aen, W_men, b_men, Wa_ih, Wa_hh, ba_ih, ba_hh, Wm_ih, Wm_hh, bm_ih, bm_hh, Wq, Wk, Wv, Wo, bo, Wc_a, bc_a, Wc_m, bc_m, W1r, b1r, W2r, b2r, W_oa, b_oa, W_of, b_of):
    raise NotImplementedError("write your pallas kernel here")


# speedup vs baseline: 1.3263x; 1.3263x over previous
import functools
import numpy as np
import jax
import jax.numpy as jnp
from jax import lax
from jax.experimental import pallas as pl
from jax.experimental.pallas import tpu as pltpu
from jax.experimental.pallas import tpu_sc as plsc

N_ATOM = 10000; E_ATOM = 160000; N_MOTIF = 2500; E_MOTIF = 20000; G = 512; H = 16
NA_P = 10240; NM_P = 2560; NV = NA_P + NM_P
EA_P = 163840; EM_P = 20480; EV = EA_P + EM_P
NW = 32
CHUNK = 128
EPW = EV // NW
NCH = EPW // CHUNK
APW = NA_P // NW
ACH = 64
ANC = APW // ACH
EB = 1024
PREC = lax.Precision.HIGHEST
f32 = jnp.float32
i32 = jnp.int32


def _dot(a, b):
    return lax.dot_general(a, b, (((a.ndim - 1,), (0,)), ((), ())), precision=PREC)


def _tp(x):
    e = jnp.eye(x.shape[1], dtype=x.dtype)
    return lax.dot_general(e, x, (((1,), (1,)), ((), ())), precision=PREC)


def _mlp_body(x_ref, w1_ref, b1_ref, w2_ref, b2_ref, o_ref):
    t = jnp.maximum(_dot(x_ref[...], w1_ref[0]) + b1_ref[0], 0.0)
    o_ref[...] = jnp.maximum(_dot(t, w2_ref[0]) + b2_ref[0], 0.0)


def _edge_body(ef_ref, us_ref, w_ref, b_ref, bt_ref, r2_ref, o_ref):
    ef = ef_ref[...]
    us = us_ref[...]
    A = _dot(ef, w_ref[0]) + b_ref[0]
    uexp = jnp.concatenate([us] * 16, axis=1)
    o_ref[...] = _dot(A * uexp, r2_ref[...]) + _dot(us, bt_ref[0])


def _gru(x, h, wir, wiz, win, whr, whz, whn, bir, biz, bin_, bhr, bhz, bhn):
    r = jax.nn.sigmoid(_dot(x, wir) + bir + _dot(h, whr) + bhr)
    z = jax.nn.sigmoid(_dot(x, wiz) + biz + _dot(h, whz) + bhz)
    n = jnp.tanh(_dot(x, win) + bin_ + r * (_dot(h, whn) + bhn))
    return (1.0 - z) * n + z * h


def _motif_body(h_ref, p_ref, aug_ref, wq_ref, wk_ref, wv_ref, wo_ref, bo_ref,
                r4_ref, e4_ref, wih_ref, whh_ref, bih_ref, bhh_ref, o_ref):
    h = h_ref[...]
    pre = (p_ref[0] + p_ref[1])[:N_MOTIF]
    aug = (aug_ref[0] + aug_ref[1])[:N_MOTIF]
    Q = _dot(h, wq_ref[...])
    k0 = _dot(aug, wk_ref[...]); k1 = _dot(pre, wk_ref[...])
    v0 = _dot(aug, wv_ref[...]); v1 = _dot(pre, wv_ref[...])
    r4 = r4_ref[...]
    s0 = _dot(Q * k0, r4) * 0.25
    s1 = _dot(Q * k1, r4) * 0.25
    m = jnp.maximum(s0, s1)
    e0 = jnp.exp(s0 - m); e1 = jnp.exp(s1 - m)
    det = e0 + e1
    e4 = e4_ref[...]
    mo = _dot(e0 / det, e4) * v0 + _dot(e1 / det, e4) * v1
    x = _dot(mo, wo_ref[...]) + bo_ref[...]
    wih = wih_ref[...]; whh = whh_ref[...]; bih = bih_ref[...]; bhh = bhh_ref[...]
    o_ref[...] = _gru(x, h,
                      wih[:, :16], wih[:, 16:32], wih[:, 32:],
                      whh[:, :16], whh[:, 16:32], whh[:, 32:],
                      bih[:, :16], bih[:, 16:32], bih[:, 32:],
                      bhh[:, :16], bhh[:, 16:32], bhh[:, 32:])


def _agru_body(x_ref, h_ref, wih_ref, whh_ref, bih_ref, bhh_ref, o_ref):
    wih = wih_ref[...]; whh = whh_ref[...]; bih = bih_ref[...]; bhh = bhh_ref[...]
    o_ref[...] = _gru(x_ref[...], h_ref[...],
                      wih[:, :16], wih[:, 16:32], wih[:, 32:],
                      whh[:, :16], whh[:, 16:32], whh[:, 32:],
                      bih[:, :16], bih[:, 16:32], bih[:, 32:],
                      bhh[:, :16], bhh[:, 16:32], bhh[:, 32:])


def _readout_body(h_ref, ids_ref, osum_ref, omax_ref):
    i = pl.program_id(0)

    @pl.when(i == 0)
    def _():
        osum_ref[...] = jnp.zeros_like(osum_ref)
        omax_ref[...] = jnp.full_like(omax_ref, -jnp.inf)

    h = h_ref[...]
    ids = ids_ref[0]
    mask = ids == lax.broadcasted_iota(i32, (G, h.shape[0]), 0)
    osum_ref[...] += _dot(mask.astype(f32), h)
    hT = _tp(h)
    neg = jnp.full((), -jnp.inf, f32)
    cols = []
    for fidx in range(16):
        t = jnp.where(mask, hT[fidx:fidx + 1, :], neg)
        cols.append(jnp.max(t, axis=1)[:, None])
    omax_ref[...] = jnp.maximum(omax_ref[...], jnp.concatenate(cols, axis=1))


def _head_body(asum_ref, amax_ref, msum_ref, mmax_ref,
               wca_ref, bca_ref, wcm_ref, bcm_ref,
               w1_ref, b1_ref, w2_ref, b2_ref,
               woa_ref, boa_ref, wof_ref, bof_ref, pa_ref, pm_ref):
    amax = amax_ref[...]
    mmax = mmax_ref[...]
    amax = jnp.where(jnp.isfinite(amax), amax, 0.0)
    mmax = jnp.where(jnp.isfinite(mmax), mmax, 0.0)
    arep = jnp.concatenate([asum_ref[...], amax], axis=1)
    mrep = jnp.concatenate([msum_ref[...], mmax], axis=1)
    a_sq = jnp.sum(arep * wca_ref[...], axis=1, keepdims=True) + bca_ref[...]
    m_sq = jnp.sum(mrep * wcm_ref[...], axis=1, keepdims=True) + bcm_ref[...]
    joint = jnp.concatenate([a_sq, m_sq], axis=1)
    t1 = _dot(joint, w1_ref[...]) + b1_ref[...]
    w = jax.nn.sigmoid(_dot(t1, w2_ref[...]) + b2_ref[...])
    pa_ref[...] = jnp.sum(arep * w[:, 0:1] * woa_ref[...], axis=1, keepdims=True) + boa_ref[...]
    pm_ref[...] = jnp.sum(mrep * w[:, 1:2] * wof_ref[...], axis=1, keepdims=True) + bof_ref[...]


def _sc_gather_body(uv, idx, out, idx_v, rows_v, sem):
    w = lax.axis_index("s") * 2 + lax.axis_index("c")

    def chunk(t, carry):
        base = w * EPW + t * CHUNK
        pltpu.sync_copy(idx.at[pl.ds(base, CHUNK)], idx_v)
        pltpu.async_copy(uv.at[idx_v], rows_v, sem).wait()
        pltpu.sync_copy(rows_v, out.at[pl.ds(base, CHUNK)])
        return carry

    lax.fori_loop(0, NCH, chunk, 0)


def _sc_scatter_body(msg, dst, zrows, out, dst_v, rows_v, acc):
    c = lax.axis_index("c")
    s = lax.axis_index("s")
    w = s * 2 + c
    stripe = NV // 16
    pltpu.sync_copy(zrows.at[pl.ds(s * stripe, stripe)], acc.at[pl.ds(s * stripe, stripe)])
    plsc.subcore_barrier()

    def chunk(t, carry):
        base = w * EPW + t * CHUNK
        pltpu.sync_copy(dst.at[pl.ds(base, CHUNK)], dst_v)
        pltpu.sync_copy(msg.at[pl.ds(base, CHUNK)], rows_v)
        pltpu.sync_copy(rows_v, acc.at[dst_v], add=True)
        return carry

    lax.fori_loop(0, NCH, chunk, 0)
    plsc.subcore_barrier()
    pltpu.sync_copy(acc.at[pl.ds(s * stripe, stripe)], out.at[c].at[pl.ds(s * stripe, stripe)])


def _sc_a2f_body(p, a2fd, zrows, uam_out, aug_out, idx_v, r0, r1, rs, acc):
    c = lax.axis_index("c")
    s = lax.axis_index("s")
    w = s * 2 + c
    stripe = NM_P // 16
    pltpu.sync_copy(zrows.at[pl.ds(s * stripe, stripe)], acc.at[pl.ds(s * stripe, stripe)])
    plsc.subcore_barrier()

    def chunk(t, carry):
        base = w * APW + t * ACH
        pltpu.sync_copy(a2fd.at[pl.ds(base, ACH)], idx_v)
        pltpu.sync_copy(p.at[0].at[pl.ds(base, ACH)], r0)
        pltpu.sync_copy(p.at[1].at[pl.ds(base, ACH)], r1)

        def row(i, cc):
            rs[i] = r0[i] + r1[i]
            return cc

        lax.fori_loop(0, ACH, row, 0)
        pltpu.sync_copy(rs, uam_out.at[pl.ds(base, ACH)])
        pltpu.sync_copy(rs, acc.at[idx_v], add=True)
        return carry

    lax.fori_loop(0, ANC, chunk, 0)
    plsc.subcore_barrier()
    pltpu.sync_copy(acc.at[pl.ds(s * stripe, stripe)], aug_out.at[c].at[pl.ds(s * stripe, stripe)])


def kernel(af, bf, fnf, fef, atom_edge_index, motif_edge_index, a2f_src, a2f_dst,
           atom_graph_ids, motif_graph_ids, labels,
           W_ae1, b_ae1, W_ae2, b_ae2, W_me1, b_me1, W_me2, b_me2,
           W_aen, b_aen, W_men, b_men,
           Wa_ih, Wa_hh, ba_ih, ba_hh, Wm_ih, Wm_hh, bm_ih, bm_hh,
           Wq, Wk, Wv, Wo, bo, Wc_a, bc_a, Wc_m, bc_m,
           W1r, b1r, W2r, b2r, W_oa, b_oa, W_of, b_of):
    W1s = jnp.stack([W_ae1, W_me1])
    B1s = jnp.stack([b_ae1, b_me1]).reshape(2, 1, 16)
    W2s = jnp.stack([W_ae2, W_me2])
    B2s = jnp.stack([b_ae2, b_me2]).reshape(2, 1, 16)
    Wes = jnp.stack([W_aen, W_men])
    Bes = jnp.stack([b_aen, b_men]).reshape(2, 1, 256)
    BTs = jnp.stack([b_aen.reshape(16, 16).T, b_men.reshape(16, 16).T])
    R2 = jnp.asarray(np.kron(np.eye(16, dtype=np.float32), np.ones((16, 1), np.float32)))
    R4 = jnp.asarray(np.kron(np.eye(4, dtype=np.float32), np.ones((4, 1), np.float32)))
    E4 = jnp.asarray(np.kron(np.eye(4, dtype=np.float32), np.ones((1, 4), np.float32)))
    bih_m = bm_ih.reshape(1, 48); bhh_m = bm_hh.reshape(1, 48)
    bih_a = ba_ih.reshape(1, 48); bhh_a = ba_hh.reshape(1, 48)

    asrc, adst = atom_edge_index[0], atom_edge_index[1]
    msrc, mdst = motif_edge_index[0], motif_edge_index[1]
    IDX = jnp.concatenate([asrc, jnp.zeros((EA_P - E_ATOM,), i32),
                           msrc + NA_P, jnp.full((EM_P - E_MOTIF,), NA_P, i32)])
    DST = jnp.concatenate([adst, jnp.full((EA_P - E_ATOM,), N_ATOM, i32),
                           mdst + NA_P, jnp.full((EM_P - E_MOTIF,), NA_P + N_MOTIF, i32)])
    A2FD = jnp.concatenate([a2f_dst, jnp.full((NA_P - N_ATOM,), N_MOTIF, i32)])
    EF = jnp.concatenate([bf, jnp.zeros((EA_P - E_ATOM, 16), f32),
                          fef, jnp.zeros((EM_P - E_MOTIF, 16), f32)], axis=0)
    ZR = jnp.zeros((NV, 16), f32)
    gids_a = atom_graph_ids.reshape(10, 1, 1000)
    gids_m = motif_graph_ids.reshape(1, 1, 2500)

    mesh = plsc.VectorSubcoreMesh(core_axis_name="c", subcore_axis_name="s")

    sc_params = pltpu.CompilerParams(use_tc_tiling_on_sc=False)

    sc_gather = functools.partial(
        pl.kernel,
        out_type=jax.ShapeDtypeStruct((EV, 16), f32),
        mesh=mesh,
        compiler_params=sc_params,
        scratch_types=[pltpu.VMEM((CHUNK,), i32), pltpu.VMEM((CHUNK, 16), f32),
                       pltpu.SemaphoreType.DMA],
    )(_sc_gather_body)

    sc_scatter = functools.partial(
        pl.kernel,
        out_type=jax.ShapeDtypeStruct((2, NV, 16), f32),
        mesh=mesh,
        compiler_params=sc_params,
        scratch_types=[pltpu.VMEM((CHUNK,), i32), pltpu.VMEM((CHUNK, 16), f32),
                       pltpu.VMEM_SHARED((NV, 16), f32)],
    )(_sc_scatter_body)

    sc_a2f = functools.partial(
        pl.kernel,
        out_type=(jax.ShapeDtypeStruct((NA_P, 16), f32),
                  jax.ShapeDtypeStruct((2, NM_P, 16), f32)),
        mesh=mesh,
        compiler_params=sc_params,
        scratch_types=[pltpu.VMEM((ACH,), i32), pltpu.VMEM((ACH, 16), f32),
                       pltpu.VMEM((ACH, 16), f32), pltpu.VMEM((ACH, 16), f32),
                       pltpu.VMEM_SHARED((NM_P, 16), f32)],
    )(_sc_a2f_body)

    X = jnp.concatenate([af, jnp.zeros((NA_P - N_ATOM, 128), f32),
                         fnf, jnp.zeros((NM_P - N_MOTIF, 128), f32)], axis=0)
    UF = pl.pallas_call(
        _mlp_body,
        grid=(20,),
        in_specs=[pl.BlockSpec((640, 128), lambda i: (i, 0)),
                  pl.BlockSpec((1, 128, 16), lambda i: (i // 16, 0, 0)),
                  pl.BlockSpec((1, 1, 16), lambda i: (i // 16, 0, 0)),
                  pl.BlockSpec((1, 16, 16), lambda i: (i // 16, 0, 0)),
                  pl.BlockSpec((1, 1, 16), lambda i: (i // 16, 0, 0))],
        out_specs=pl.BlockSpec((640, 16), lambda i: (i, 0)),
        out_shape=jax.ShapeDtypeStruct((NV, 16), f32),
    )(X, W1s, B1s, W2s, B2s)
    uaf = UF[:N_ATOM]
    ufnf = UF[NA_P:NA_P + N_MOTIF]

    edge_call = pl.pallas_call(
        _edge_body,
        grid=(EV // EB,),
        in_specs=[pl.BlockSpec((EB, 16), lambda i: (i, 0)),
                  pl.BlockSpec((EB, 16), lambda i: (i, 0)),
                  pl.BlockSpec((1, 16, 256), lambda i: (i // (EA_P // EB), 0, 0)),
                  pl.BlockSpec((1, 1, 256), lambda i: (i // (EA_P // EB), 0, 0)),
                  pl.BlockSpec((1, 16, 16), lambda i: (i // (EA_P // EB), 0, 0)),
                  pl.BlockSpec((256, 16), lambda i: (0, 0))],
        out_specs=pl.BlockSpec((EB, 16), lambda i: (i, 0)),
        out_shape=jax.ShapeDtypeStruct((EV, 16), f32),
    )

    motif_call = pl.pallas_call(
        _motif_body,
        grid=(1,),
        in_specs=[pl.BlockSpec((N_MOTIF, 16), lambda i: (0, 0)),
                  pl.BlockSpec((2, NM_P, 16), lambda i: (0, NA_P // NM_P, 0)),
                  pl.BlockSpec((2, NM_P, 16), lambda i: (0, 0, 0)),
                  pl.BlockSpec((16, 16), lambda i: (0, 0)),
                  pl.BlockSpec((16, 16), lambda i: (0, 0)),
                  pl.BlockSpec((16, 16), lambda i: (0, 0)),
                  pl.BlockSpec((16, 16), lambda i: (0, 0)),
                  pl.BlockSpec((1, 16), lambda i: (0, 0)),
                  pl.BlockSpec((16, 4), lambda i: (0, 0)),
                  pl.BlockSpec((4, 16), lambda i: (0, 0)),
                  pl.BlockSpec((16, 48), lambda i: (0, 0)),
                  pl.BlockSpec((16, 48), lambda i: (0, 0)),
                  pl.BlockSpec((1, 48), lambda i: (0, 0)),
                  pl.BlockSpec((1, 48), lambda i: (0, 0))],
        out_specs=pl.BlockSpec((N_MOTIF, 16), lambda i: (0, 0)),
        out_shape=jax.ShapeDtypeStruct((N_MOTIF, 16), f32),
    )

    agru_call = pl.pallas_call(
        _agru_body,
        grid=(10,),
        in_specs=[pl.BlockSpec((1000, 16), lambda i: (i, 0)),
                  pl.BlockSpec((1000, 16), lambda i: (i, 0)),
                  pl.BlockSpec((16, 48), lambda i: (0, 0)),
                  pl.BlockSpec((16, 48), lambda i: (0, 0)),
                  pl.BlockSpec((1, 48), lambda i: (0, 0)),
                  pl.BlockSpec((1, 48), lambda i: (0, 0))],
        out_specs=pl.BlockSpec((1000, 16), lambda i: (i, 0)),
        out_shape=jax.ShapeDtypeStruct((N_ATOM, 16), f32),
    )

    for _ in range(2):
        UV = jnp.concatenate([uaf, jnp.zeros((NA_P - N_ATOM, 16), f32),
                              ufnf, jnp.zeros((NM_P - N_MOTIF, 16), f32)], axis=0)
        UALL = sc_gather(UV, IDX)
        MSG = edge_call(EF, UALL, Wes, Bes, BTs, R2)
        P = sc_scatter(MSG, DST, ZR)
        uam_full, AUGP = sc_a2f(P, A2FD, ZR)
        ufnf = motif_call(ufnf, P, AUGP, Wq, Wk, Wv, Wo, bo.reshape(1, 16),
                          R4, E4, Wm_ih, Wm_hh, bih_m, bhh_m)
        uaf = agru_call(uam_full, uaf, Wa_ih, Wa_hh, bih_a, bhh_a)

    asum, amax = pl.pallas_call(
        _readout_body,
        grid=(10,),
        in_specs=[pl.BlockSpec((1000, 16), lambda i: (i, 0)),
                  pl.BlockSpec((1, 1, 1000), lambda i: (i, 0, 0))],
        out_specs=[pl.BlockSpec((G, 16), lambda i: (0, 0)),
                   pl.BlockSpec((G, 16), lambda i: (0, 0))],
        out_shape=(jax.ShapeDtypeStruct((G, 16), f32), jax.ShapeDtypeStruct((G, 16), f32)),
    )(uaf, gids_a)
    msum, mmax = pl.pallas_call(
        _readout_body,
        grid=(1,),
        in_specs=[pl.BlockSpec((2500, 16), lambda i: (i, 0)),
                  pl.BlockSpec((1, 1, 2500), lambda i: (i, 0, 0))],
        out_specs=[pl.BlockSpec((G, 16), lambda i: (0, 0)),
                   pl.BlockSpec((G, 16), lambda i: (0, 0))],
        out_shape=(jax.ShapeDtypeStruct((G, 16), f32), jax.ShapeDtypeStruct((G, 16), f32)),
    )(ufnf, gids_m)

    full = lambda shape: pl.BlockSpec(shape, lambda: tuple(0 for _ in shape))
    pred_a, pred_m = pl.pallas_call(
        _head_body,
        in_specs=[full((G, 16)), full((G, 16)), full((G, 16)), full((G, 16)),
                  full((1, 32)), full((1, 1)), full((1, 32)), full((1, 1)),
                  full((2, 64)), full((1, 64)), full((64, 2)), full((1, 2)),
                  full((1, 32)), full((1, 1)), full((1, 32)), full((1, 1))],
        out_specs=[full((G, 1)), full((G, 1))],
        out_shape=(jax.ShapeDtypeStruct((G, 1), f32), jax.ShapeDtypeStruct((G, 1), f32)),
    )(asum, amax, msum, mmax,
      Wc_a.T, bc_a.reshape(1, 1), Wc_m.T, bc_m.reshape(1, 1),
      W1r, b1r.reshape(1, 64), W2r, b2r.reshape(1, 2),
      W_oa.T, b_oa.reshape(1, 1), W_of.T, b_of.reshape(1, 1))
    return pred_a, pred_m


# speedup vs baseline: 2.1287x; 1.6049x over previous
import functools
import numpy as np
import jax
import jax.numpy as jnp
from jax import lax
from jax.experimental import pallas as pl
from jax.experimental.pallas import tpu as pltpu
from jax.experimental.pallas import tpu_sc as plsc

N_ATOM = 10000; E_ATOM = 160000; N_MOTIF = 2500; E_MOTIF = 20000; G = 512; H = 16
NA_P = 10240; NM_P = 2560; NV = NA_P + NM_P
EA_P = 163840; EM_P = 20480; EV = EA_P + EM_P
NW = 32
CHUNK = 128
EPW = EV // NW
NCH = EPW // CHUNK
APW = NA_P // NW
ACH = 64
ANC = APW // ACH
EB = 1024
PREC = lax.Precision.DEFAULT
f32 = jnp.float32
i32 = jnp.int32


def _dot(a, b):
    return lax.dot_general(a, b, (((a.ndim - 1,), (0,)), ((), ())), precision=PREC)


def _tp(x):
    e = jnp.eye(x.shape[1], dtype=x.dtype)
    return lax.dot_general(e, x, (((1,), (1,)), ((), ())), precision=PREC)


def _mlp_body(x_ref, w1_ref, b1_ref, w2_ref, b2_ref, o_ref):
    t = jnp.maximum(_dot(x_ref[...], w1_ref[0]) + b1_ref[0], 0.0)
    o_ref[...] = jnp.maximum(_dot(t, w2_ref[0]) + b2_ref[0], 0.0)


def _edge_body(ef_ref, us_ref, w_ref, b_ref, bt_ref, r2_ref, o_ref):
    ef = ef_ref[...]
    us = us_ref[...]
    A = _dot(ef, w_ref[0]) + b_ref[0]
    uexp = jnp.concatenate([us] * 16, axis=1)
    o_ref[...] = _dot(A * uexp, r2_ref[...]) + _dot(us, bt_ref[0])


def _gru(x, h, wir, wiz, win, whr, whz, whn, bir, biz, bin_, bhr, bhz, bhn):
    r = jax.nn.sigmoid(_dot(x, wir) + bir + _dot(h, whr) + bhr)
    z = jax.nn.sigmoid(_dot(x, wiz) + biz + _dot(h, whz) + bhz)
    n = jnp.tanh(_dot(x, win) + bin_ + r * (_dot(h, whn) + bhn))
    return (1.0 - z) * n + z * h


def _motif_body(h_ref, p_ref, aug_ref, wq_ref, wk_ref, wv_ref, wo_ref, bo_ref,
                r4_ref, e4_ref, wih_ref, whh_ref, bih_ref, bhh_ref, o_ref):
    h = h_ref[...]
    pre = (p_ref[0] + p_ref[1])[:N_MOTIF]
    aug = (aug_ref[0] + aug_ref[1])[:N_MOTIF]
    Q = _dot(h, wq_ref[...])
    k0 = _dot(aug, wk_ref[...]); k1 = _dot(pre, wk_ref[...])
    v0 = _dot(aug, wv_ref[...]); v1 = _dot(pre, wv_ref[...])
    r4 = r4_ref[...]
    s0 = _dot(Q * k0, r4) * 0.25
    s1 = _dot(Q * k1, r4) * 0.25
    m = jnp.maximum(s0, s1)
    e0 = jnp.exp(s0 - m); e1 = jnp.exp(s1 - m)
    det = e0 + e1
    e4 = e4_ref[...]
    mo = _dot(e0 / det, e4) * v0 + _dot(e1 / det, e4) * v1
    x = _dot(mo, wo_ref[...]) + bo_ref[...]
    wih = wih_ref[...]; whh = whh_ref[...]; bih = bih_ref[...]; bhh = bhh_ref[...]
    o_ref[...] = _gru(x, h,
                      wih[:, :16], wih[:, 16:32], wih[:, 32:],
                      whh[:, :16], whh[:, 16:32], whh[:, 32:],
                      bih[:, :16], bih[:, 16:32], bih[:, 32:],
                      bhh[:, :16], bhh[:, 16:32], bhh[:, 32:])


def _agru_body(x_ref, h_ref, wih_ref, whh_ref, bih_ref, bhh_ref, o_ref):
    wih = wih_ref[...]; whh = whh_ref[...]; bih = bih_ref[...]; bhh = bhh_ref[...]
    o_ref[...] = _gru(x_ref[...], h_ref[...],
                      wih[:, :16], wih[:, 16:32], wih[:, 32:],
                      whh[:, :16], whh[:, 16:32], whh[:, 32:],
                      bih[:, :16], bih[:, 16:32], bih[:, 32:],
                      bhh[:, :16], bhh[:, 16:32], bhh[:, 32:])


def _readout_body(h_ref, ids_ref, osum_ref, omax_ref):
    i = pl.program_id(0)

    @pl.when(i == 0)
    def _():
        osum_ref[...] = jnp.zeros_like(osum_ref)
        omax_ref[...] = jnp.full_like(omax_ref, -jnp.inf)

    h = h_ref[...]
    ids = ids_ref[0]
    mask = ids == lax.broadcasted_iota(i32, (G, h.shape[0]), 0)
    osum_ref[...] += _dot(mask.astype(f32), h)
    hT = _tp(h)
    neg = jnp.full((), -jnp.inf, f32)
    cols = []
    for fidx in range(16):
        t = jnp.where(mask, hT[fidx:fidx + 1, :], neg)
        cols.append(jnp.max(t, axis=1)[:, None])
    omax_ref[...] = jnp.maximum(omax_ref[...], jnp.concatenate(cols, axis=1))


def _head_body(asum_ref, amax_ref, msum_ref, mmax_ref,
               wca_ref, bca_ref, wcm_ref, bcm_ref,
               w1_ref, b1_ref, w2_ref, b2_ref,
               woa_ref, boa_ref, wof_ref, bof_ref, pa_ref, pm_ref):
    amax = amax_ref[...]
    mmax = mmax_ref[...]
    amax = jnp.where(jnp.isfinite(amax), amax, 0.0)
    mmax = jnp.where(jnp.isfinite(mmax), mmax, 0.0)
    arep = jnp.concatenate([asum_ref[...], amax], axis=1)
    mrep = jnp.concatenate([msum_ref[...], mmax], axis=1)
    a_sq = jnp.sum(arep * wca_ref[...], axis=1, keepdims=True) + bca_ref[...]
    m_sq = jnp.sum(mrep * wcm_ref[...], axis=1, keepdims=True) + bcm_ref[...]
    joint = jnp.concatenate([a_sq, m_sq], axis=1)
    t1 = _dot(joint, w1_ref[...]) + b1_ref[...]
    w = jax.nn.sigmoid(_dot(t1, w2_ref[...]) + b2_ref[...])
    pa_ref[...] = jnp.sum(arep * w[:, 0:1] * woa_ref[...], axis=1, keepdims=True) + boa_ref[...]
    pm_ref[...] = jnp.sum(mrep * w[:, 1:2] * wof_ref[...], axis=1, keepdims=True) + bof_ref[...]


def _sc_gather_body(uv, idx, out, idx_v, rows_v, sem):
    w = lax.axis_index("s") * 2 + lax.axis_index("c")

    def chunk(t, carry):
        base = w * EPW + t * CHUNK
        pltpu.sync_copy(idx.at[pl.ds(base, CHUNK)], idx_v)
        pltpu.async_copy(uv.at[idx_v], rows_v, sem).wait()
        pltpu.sync_copy(rows_v, out.at[pl.ds(base, CHUNK)])
        return carry

    lax.fori_loop(0, NCH, chunk, 0)


def _sc_scatter_body(msg, dst, zrows, out, dst_v, rows_v, acc):
    c = lax.axis_index("c")
    s = lax.axis_index("s")
    w = s * 2 + c
    stripe = NV // 16
    pltpu.sync_copy(zrows.at[pl.ds(s * stripe, stripe)], acc.at[pl.ds(s * stripe, stripe)])
    plsc.subcore_barrier()

    def chunk(t, carry):
        base = w * EPW + t * CHUNK
        pltpu.sync_copy(dst.at[pl.ds(base, CHUNK)], dst_v)
        pltpu.sync_copy(msg.at[pl.ds(base, CHUNK)], rows_v)
        pltpu.sync_copy(rows_v, acc.at[dst_v], add=True)
        return carry

    lax.fori_loop(0, NCH, chunk, 0)
    plsc.subcore_barrier()
    pltpu.sync_copy(acc.at[pl.ds(s * stripe, stripe)], out.at[c].at[pl.ds(s * stripe, stripe)])


def _sc_a2f_body(p, a2fd, zrows, uam_out, aug_out, idx_v, r0, r1, rs, acc):
    c = lax.axis_index("c")
    s = lax.axis_index("s")
    w = s * 2 + c
    stripe = NM_P // 16
    pltpu.sync_copy(zrows.at[pl.ds(s * stripe, stripe)], acc.at[pl.ds(s * stripe, stripe)])
    plsc.subcore_barrier()

    def chunk(t, carry):
        base = w * APW + t * ACH
        pltpu.sync_copy(a2fd.at[pl.ds(base, ACH)], idx_v)
        pltpu.sync_copy(p.at[0].at[pl.ds(base, ACH)], r0)
        pltpu.sync_copy(p.at[1].at[pl.ds(base, ACH)], r1)

        def row(i, cc):
            rs[i] = r0[i] + r1[i]
            return cc

        lax.fori_loop(0, ACH, row, 0)
        pltpu.sync_copy(rs, uam_out.at[pl.ds(base, ACH)])
        pltpu.sync_copy(rs, acc.at[idx_v], add=True)
        return carry

    lax.fori_loop(0, ANC, chunk, 0)
    plsc.subcore_barrier()
    pltpu.sync_copy(acc.at[pl.ds(s * stripe, stripe)], aug_out.at[c].at[pl.ds(s * stripe, stripe)])


def kernel(af, bf, fnf, fef, atom_edge_index, motif_edge_index, a2f_src, a2f_dst,
           atom_graph_ids, motif_graph_ids, labels,
           W_ae1, b_ae1, W_ae2, b_ae2, W_me1, b_me1, W_me2, b_me2,
           W_aen, b_aen, W_men, b_men,
           Wa_ih, Wa_hh, ba_ih, ba_hh, Wm_ih, Wm_hh, bm_ih, bm_hh,
           Wq, Wk, Wv, Wo, bo, Wc_a, bc_a, Wc_m, bc_m,
           W1r, b1r, W2r, b2r, W_oa, b_oa, W_of, b_of):
    W1s = jnp.stack([W_ae1, W_me1])
    B1s = jnp.stack([b_ae1, b_me1]).reshape(2, 1, 16)
    W2s = jnp.stack([W_ae2, W_me2])
    B2s = jnp.stack([b_ae2, b_me2]).reshape(2, 1, 16)
    Wes = jnp.stack([W_aen, W_men])
    Bes = jnp.stack([b_aen, b_men]).reshape(2, 1, 256)
    BTs = jnp.stack([b_aen.reshape(16, 16).T, b_men.reshape(16, 16).T])
    R2 = jnp.asarray(np.kron(np.eye(16, dtype=np.float32), np.ones((16, 1), np.float32)))
    R4 = jnp.asarray(np.kron(np.eye(4, dtype=np.float32), np.ones((4, 1), np.float32)))
    E4 = jnp.asarray(np.kron(np.eye(4, dtype=np.float32), np.ones((1, 4), np.float32)))
    bih_m = bm_ih.reshape(1, 48); bhh_m = bm_hh.reshape(1, 48)
    bih_a = ba_ih.reshape(1, 48); bhh_a = ba_hh.reshape(1, 48)

    asrc, adst = atom_edge_index[0], atom_edge_index[1]
    msrc, mdst = motif_edge_index[0], motif_edge_index[1]
    IDX = jnp.concatenate([asrc, jnp.zeros((EA_P - E_ATOM,), i32),
                           msrc + NA_P, jnp.full((EM_P - E_MOTIF,), NA_P, i32)])
    DST = jnp.concatenate([adst, jnp.full((EA_P - E_ATOM,), N_ATOM, i32),
                           mdst + NA_P, jnp.full((EM_P - E_MOTIF,), NA_P + N_MOTIF, i32)])
    A2FD = jnp.concatenate([a2f_dst, jnp.full((NA_P - N_ATOM,), N_MOTIF, i32)])
    EF = jnp.concatenate([bf, jnp.zeros((EA_P - E_ATOM, 16), f32),
                          fef, jnp.zeros((EM_P - E_MOTIF, 16), f32)], axis=0)
    ZR = jnp.zeros((NV, 16), f32)
    gids_a = atom_graph_ids.reshape(10, 1, 1000)
    gids_m = motif_graph_ids.reshape(1, 1, 2500)

    mesh = plsc.VectorSubcoreMesh(core_axis_name="c", subcore_axis_name="s")

    sc_params = pltpu.CompilerParams(use_tc_tiling_on_sc=False)

    sc_gather = functools.partial(
        pl.kernel,
        out_type=jax.ShapeDtypeStruct((EV, 16), f32),
        mesh=mesh,
        compiler_params=sc_params,
        scratch_types=[pltpu.VMEM((CHUNK,), i32), pltpu.VMEM((CHUNK, 16), f32),
                       pltpu.SemaphoreType.DMA],
    )(_sc_gather_body)

    sc_scatter = functools.partial(
        pl.kernel,
        out_type=jax.ShapeDtypeStruct((2, NV, 16), f32),
        mesh=mesh,
        compiler_params=sc_params,
        scratch_types=[pltpu.VMEM((CHUNK,), i32), pltpu.VMEM((CHUNK, 16), f32),
                       pltpu.VMEM_SHARED((NV, 16), f32)],
    )(_sc_scatter_body)

    sc_a2f = functools.partial(
        pl.kernel,
        out_type=(jax.ShapeDtypeStruct((NA_P, 16), f32),
                  jax.ShapeDtypeStruct((2, NM_P, 16), f32)),
        mesh=mesh,
        compiler_params=sc_params,
        scratch_types=[pltpu.VMEM((ACH,), i32), pltpu.VMEM((ACH, 16), f32),
                       pltpu.VMEM((ACH, 16), f32), pltpu.VMEM((ACH, 16), f32),
                       pltpu.VMEM_SHARED((NM_P, 16), f32)],
    )(_sc_a2f_body)

    X = jnp.concatenate([af, jnp.zeros((NA_P - N_ATOM, 128), f32),
                         fnf, jnp.zeros((NM_P - N_MOTIF, 128), f32)], axis=0)
    UF = pl.pallas_call(
        _mlp_body,
        grid=(20,),
        in_specs=[pl.BlockSpec((640, 128), lambda i: (i, 0)),
                  pl.BlockSpec((1, 128, 16), lambda i: (i // 16, 0, 0)),
                  pl.BlockSpec((1, 1, 16), lambda i: (i // 16, 0, 0)),
                  pl.BlockSpec((1, 16, 16), lambda i: (i // 16, 0, 0)),
                  pl.BlockSpec((1, 1, 16), lambda i: (i // 16, 0, 0))],
        out_specs=pl.BlockSpec((640, 16), lambda i: (i, 0)),
        out_shape=jax.ShapeDtypeStruct((NV, 16), f32),
    )(X, W1s, B1s, W2s, B2s)
    uaf = UF[:N_ATOM]
    ufnf = UF[NA_P:NA_P + N_MOTIF]

    edge_call = pl.pallas_call(
        _edge_body,
        grid=(EV // EB,),
        in_specs=[pl.BlockSpec((EB, 16), lambda i: (i, 0)),
                  pl.BlockSpec((EB, 16), lambda i: (i, 0)),
                  pl.BlockSpec((1, 16, 256), lambda i: (i // (EA_P // EB), 0, 0)),
                  pl.BlockSpec((1, 1, 256), lambda i: (i // (EA_P // EB), 0, 0)),
                  pl.BlockSpec((1, 16, 16), lambda i: (i // (EA_P // EB), 0, 0)),
                  pl.BlockSpec((256, 16), lambda i: (0, 0))],
        out_specs=pl.BlockSpec((EB, 16), lambda i: (i, 0)),
        out_shape=jax.ShapeDtypeStruct((EV, 16), f32),
    )

    motif_call = pl.pallas_call(
        _motif_body,
        grid=(1,),
        in_specs=[pl.BlockSpec((N_MOTIF, 16), lambda i: (0, 0)),
                  pl.BlockSpec((2, NM_P, 16), lambda i: (0, NA_P // NM_P, 0)),
                  pl.BlockSpec((2, NM_P, 16), lambda i: (0, 0, 0)),
                  pl.BlockSpec((16, 16), lambda i: (0, 0)),
                  pl.BlockSpec((16, 16), lambda i: (0, 0)),
                  pl.BlockSpec((16, 16), lambda i: (0, 0)),
                  pl.BlockSpec((16, 16), lambda i: (0, 0)),
                  pl.BlockSpec((1, 16), lambda i: (0, 0)),
                  pl.BlockSpec((16, 4), lambda i: (0, 0)),
                  pl.BlockSpec((4, 16), lambda i: (0, 0)),
                  pl.BlockSpec((16, 48), lambda i: (0, 0)),
                  pl.BlockSpec((16, 48), lambda i: (0, 0)),
                  pl.BlockSpec((1, 48), lambda i: (0, 0)),
                  pl.BlockSpec((1, 48), lambda i: (0, 0))],
        out_specs=pl.BlockSpec((N_MOTIF, 16), lambda i: (0, 0)),
        out_shape=jax.ShapeDtypeStruct((N_MOTIF, 16), f32),
    )

    agru_call = pl.pallas_call(
        _agru_body,
        grid=(10,),
        in_specs=[pl.BlockSpec((1000, 16), lambda i: (i, 0)),
                  pl.BlockSpec((1000, 16), lambda i: (i, 0)),
                  pl.BlockSpec((16, 48), lambda i: (0, 0)),
                  pl.BlockSpec((16, 48), lambda i: (0, 0)),
                  pl.BlockSpec((1, 48), lambda i: (0, 0)),
                  pl.BlockSpec((1, 48), lambda i: (0, 0))],
        out_specs=pl.BlockSpec((1000, 16), lambda i: (i, 0)),
        out_shape=jax.ShapeDtypeStruct((N_ATOM, 16), f32),
    )

    for _ in range(2):
        UV = jnp.concatenate([uaf, jnp.zeros((NA_P - N_ATOM, 16), f32),
                              ufnf, jnp.zeros((NM_P - N_MOTIF, 16), f32)], axis=0)
        UALL = sc_gather(UV, IDX)
        MSG = edge_call(EF, UALL, Wes, Bes, BTs, R2)
        P = sc_scatter(MSG, DST, ZR)
        uam_full, AUGP = sc_a2f(P, A2FD, ZR)
        ufnf = motif_call(ufnf, P, AUGP, Wq, Wk, Wv, Wo, bo.reshape(1, 16),
                          R4, E4, Wm_ih, Wm_hh, bih_m, bhh_m)
        uaf = agru_call(uam_full, uaf, Wa_ih, Wa_hh, bih_a, bhh_a)

    asum, amax = pl.pallas_call(
        _readout_body,
        grid=(10,),
        in_specs=[pl.BlockSpec((1000, 16), lambda i: (i, 0)),
                  pl.BlockSpec((1, 1, 1000), lambda i: (i, 0, 0))],
        out_specs=[pl.BlockSpec((G, 16), lambda i: (0, 0)),
                   pl.BlockSpec((G, 16), lambda i: (0, 0))],
        out_shape=(jax.ShapeDtypeStruct((G, 16), f32), jax.ShapeDtypeStruct((G, 16), f32)),
    )(uaf, gids_a)
    msum, mmax = pl.pallas_call(
        _readout_body,
        grid=(1,),
        in_specs=[pl.BlockSpec((2500, 16), lambda i: (i, 0)),
                  pl.BlockSpec((1, 1, 2500), lambda i: (i, 0, 0))],
        out_specs=[pl.BlockSpec((G, 16), lambda i: (0, 0)),
                   pl.BlockSpec((G, 16), lambda i: (0, 0))],
        out_shape=(jax.ShapeDtypeStruct((G, 16), f32), jax.ShapeDtypeStruct((G, 16), f32)),
    )(ufnf, gids_m)

    full = lambda shape: pl.BlockSpec(shape, lambda: tuple(0 for _ in shape))
    pred_a, pred_m = pl.pallas_call(
        _head_body,
        in_specs=[full((G, 16)), full((G, 16)), full((G, 16)), full((G, 16)),
                  full((1, 32)), full((1, 1)), full((1, 32)), full((1, 1)),
                  full((2, 64)), full((1, 64)), full((64, 2)), full((1, 2)),
                  full((1, 32)), full((1, 1)), full((1, 32)), full((1, 1))],
        out_specs=[full((G, 1)), full((G, 1))],
        out_shape=(jax.ShapeDtypeStruct((G, 1), f32), jax.ShapeDtypeStruct((G, 1), f32)),
    )(asum, amax, msum, mmax,
      Wc_a.T, bc_a.reshape(1, 1), Wc_m.T, bc_m.reshape(1, 1),
      W1r, b1r.reshape(1, 64), W2r, b2r.reshape(1, 2),
      W_oa.T, b_oa.reshape(1, 1), W_of.T, b_of.reshape(1, 1))
    return pred_a, pred_m
